# baseline (speedup 1.0000x reference)
import functools

import jax
import jax.numpy as jnp
from jax import lax
from jax.experimental import pallas as pl
from jax.experimental.pallas import tpu as pltpu
from jax.experimental.pallas import tpu_sc as plsc

N = 65536
C = 12
CP = 16
NC, NS, L = 2, 16, 16
NW = NC * NS
BPW = N // NW
CH = 128
NCHUNK = BPW // CH
G = CH // L

HXY = 128
HUV = 256
BN = 512


def _sc_body(xyuv_hbm, txy_hbm, tuv_hbm, out_hbm,
             xy_v, ixy_v, iuv_v, wt_v, rxy_v, ruv_v, k0_v,
             sxy0, sxy1, suv0, suv1):
    wid = lax.axis_index("s") * NC + lax.axis_index("c")
    base = wid * BPW
    sems_xy = (sxy0, sxy1)
    sems_uv = (suv0, suv1)

    pltpu.sync_copy(xyuv_hbm.at[:, pl.ds(base, BPW)], xy_v)

    def stage(ci, b):
        o = ci * CH

        def idxb(g):
            s = g * L
            x = jnp.clip(xy_v[0, pl.ds(o + s, L)], 0.0, 1.0) * float(HXY - 1)
            y = jnp.clip(xy_v[1, pl.ds(o + s, L)], 0.0, 1.0) * float(HXY - 1)
            u = jnp.clip(xy_v[2, pl.ds(o + s, L)], 0.0, 1.0) * float(HUV - 1)
            v = jnp.clip(xy_v[3, pl.ds(o + s, L)], 0.0, 1.0) * float(HUV - 1)
            xi = jnp.minimum(x.astype(jnp.int32), HXY - 2)
            yi = jnp.minimum(y.astype(jnp.int32), HXY - 2)
            ui = jnp.minimum(u.astype(jnp.int32), HUV - 2)
            vi = jnp.minimum(v.astype(jnp.int32), HUV - 2)
            bxy = xi * HXY + yi
            buv = ui * HUV + vi
            ixy_v[b, 0, pl.ds(s, L)] = bxy
            ixy_v[b, 1, pl.ds(s, L)] = bxy + 1
            ixy_v[b, 2, pl.ds(s, L)] = bxy + HXY
            ixy_v[b, 3, pl.ds(s, L)] = bxy + (HXY + 1)
            iuv_v[b, 0, pl.ds(s, L)] = buv
            iuv_v[b, 1, pl.ds(s, L)] = buv + 1
            iuv_v[b, 2, pl.ds(s, L)] = buv + HUV
            iuv_v[b, 3, pl.ds(s, L)] = buv + (HUV + 1)
            wt_v[b, 0, pl.ds(s, L)] = x - xi.astype(jnp.float32)
            wt_v[b, 1, pl.ds(s, L)] = y - yi.astype(jnp.float32)
            wt_v[b, 2, pl.ds(s, L)] = u - ui.astype(jnp.float32)
            wt_v[b, 3, pl.ds(s, L)] = v - vi.astype(jnp.float32)
        with jax.named_scope("sc_idx"):
            plsc.parallel_loop(0, G, unroll=2)(idxb)

        with jax.named_scope("sc_fire"):
            for k in range(4):
                pltpu.async_copy(txy_hbm.at[ixy_v.at[b, k]], rxy_v.at[b, k],
                                 sems_xy[b])
                pltpu.async_copy(tuv_hbm.at[iuv_v.at[b, k]], ruv_v.at[b, k],
                                 sems_uv[b])

    def consume(ci, b):
        cb = base + ci * CH
        with jax.named_scope("sc_wait"):
            for k in range(4):
                pltpu.make_async_copy(txy_hbm.at[ixy_v.at[b, k]],
                                      rxy_v.at[b, k], sems_xy[b]).wait()
                pltpu.make_async_copy(tuv_hbm.at[iuv_v.at[b, k]],
                                      ruv_v.at[b, k], sems_uv[b]).wait()

        def ib(g):
            s = g * L
            wx = wt_v[b, 0, pl.ds(s, L)]
            wy = wt_v[b, 1, pl.ds(s, L)]
            wu = wt_v[b, 2, pl.ds(s, L)]
            wv = wt_v[b, 3, pl.ds(s, L)]
            pxy = ((1.0 - wx) * (1.0 - wy), (1.0 - wx) * wy,
                   wx * (1.0 - wy), wx * wy)
            puv = ((1.0 - wu) * (1.0 - wv), (1.0 - wu) * wv,
                   wu * (1.0 - wv), wu * wv)
            for j in range(L):
                r = s + j
                lane = jnp.full((L,), j, jnp.int32)
                fxy = sum(
                    rxy_v[b, k, r, pl.ds(0, L)]
                    * jnp.take_along_axis(pxy[k], lane, axis=0,
                                          mode="promise_in_bounds")
                    for k in range(4))
                fuv = sum(
                    ruv_v[b, k, r, pl.ds(0, L)]
                    * jnp.take_along_axis(puv[k], lane, axis=0,
                                          mode="promise_in_bounds")
                    for k in range(4))
                k0_v[r] = fxy * fuv
        with jax.named_scope("sc_interp"):
            plsc.parallel_loop(0, G, unroll=1)(ib)

        with jax.named_scope("sc_k0st"):
            pltpu.sync_copy(k0_v, out_hbm.at[pl.ds(cb, CH)])

    stage(0, 0)

    def chunk2(cj, carry):
        for b in range(2):
            ci = cj * 2 + b

            @pl.when(ci + 1 < NCHUNK)
            def _():
                stage(ci + 1, 1 - b)
            consume(ci, b)
        return carry
    lax.fori_loop(0, NCHUNK // 2, chunk2, 0)


@functools.cache
def _sc_interp():
    return functools.partial(
        pl.kernel,
        out_type=jax.ShapeDtypeStruct((N, CP), jnp.float32),
        mesh=plsc.VectorSubcoreMesh(core_axis_name="c", subcore_axis_name="s",
                                    num_cores=NC, num_subcores=NS),
        scratch_types=[
            pltpu.VMEM((4, BPW), jnp.float32),
            pltpu.VMEM((2, 4, CH), jnp.int32),
            pltpu.VMEM((2, 4, CH), jnp.int32),
            pltpu.VMEM((2, 4, CH), jnp.float32),
            pltpu.VMEM((2, 4, CH, CP), jnp.float32),
            pltpu.VMEM((2, 4, CH, CP), jnp.float32),
            pltpu.VMEM((CH, CP), jnp.float32),
            pltpu.SemaphoreType.DMA,
            pltpu.SemaphoreType.DMA,
            pltpu.SemaphoreType.DMA,
            pltpu.SemaphoreType.DMA,
        ],
        compiler_params=pltpu.CompilerParams(needs_layout_passes=False,
                                             use_tc_tiling_on_sc=False,
                                             disable_bounds_checks=True),
    )(_sc_body)


def _mlp_body(x_ref, w1_ref, b1_ref, w2_ref, b2_ref, w3_ref, b3_ref, o_ref):
    x = x_ref[...].astype(jnp.bfloat16)
    h = jnp.maximum(jnp.dot(x, w1_ref[...], preferred_element_type=jnp.float32)
                    + b1_ref[...], 0.0)
    h = jnp.maximum(jnp.dot(h.astype(jnp.bfloat16), w2_ref[...],
                            preferred_element_type=jnp.float32)
                    + b2_ref[...], 0.0)
    o = (jnp.dot(h.astype(jnp.bfloat16), w3_ref[...],
                 preferred_element_type=jnp.float32) + b3_ref[...])
    o_ref[...] = jax.nn.sigmoid(o)


_mlp = pl.pallas_call(
    _mlp_body,
    grid=(N // BN,),
    in_specs=[
        pl.BlockSpec((BN, CP), lambda i: (i, 0)),
        pl.BlockSpec((CP, 128), lambda i: (0, 0)),
        pl.BlockSpec((1, 128), lambda i: (0, 0)),
        pl.BlockSpec((128, 128), lambda i: (0, 0)),
        pl.BlockSpec((1, 128), lambda i: (0, 0)),
        pl.BlockSpec((128, 8), lambda i: (0, 0)),
        pl.BlockSpec((1, 8), lambda i: (0, 0)),
    ],
    out_specs=pl.BlockSpec((BN, 8), lambda i: (i, 0)),
    out_shape=jax.ShapeDtypeStruct((N, 8), jnp.float32),
)


def kernel(xyuv, plane_xy, plane_uv, W1, b1, W2, b2, W3, b3):
    xyuv_t = xyuv.T
    txy = jnp.pad(plane_xy, ((0, 0), (0, 0), (0, CP - C))).reshape(HXY * HXY, CP)
    tuv = jnp.pad(plane_uv, ((0, 0), (0, 0), (0, CP - C))).reshape(HUV * HUV, CP)
    k0 = _sc_interp()(xyuv_t, txy, tuv)
    w1p = jnp.pad(W1, ((0, CP - C), (0, 0))).astype(jnp.bfloat16)
    w3p = jnp.pad(W3, ((0, 0), (0, 8 - 3))).astype(jnp.bfloat16)
    b3p = jnp.pad(b3, (0, 8 - 3))
    out = _mlp(k0, w1p, b1.reshape(1, -1), W2.astype(jnp.bfloat16),
               b2.reshape(1, -1), w3p, b3p.reshape(1, -1))
    return out[:, :3]


# speedup vs baseline: 1.4594x; 1.4594x over previous
import functools

import jax
import jax.numpy as jnp
from jax import lax
from jax.experimental import pallas as pl
from jax.experimental.pallas import tpu as pltpu
from jax.experimental.pallas import tpu_sc as plsc

N = 65536
C = 12
CP = 16
NC, NS, L = 2, 16, 16
NW = NC * NS
BPW = N // NW
CH = 128
NCHUNK = BPW // CH
G = CH // L

HXY = 128
HUV = 256
BN = 4096


def _sc_body(xyuv_hbm, txy_hbm, tuv_hbm, out_hbm,
             xy_v, ixy_v, iuv_v, wt_v, rxy_v, ruv_v, k0_v,
             sxy0, sxy1, suv0, suv1):
    wid = lax.axis_index("s") * NC + lax.axis_index("c")
    base = wid * BPW
    sems_xy = (sxy0, sxy1)
    sems_uv = (suv0, suv1)

    pltpu.sync_copy(xyuv_hbm.at[:, pl.ds(base, BPW)], xy_v)

    def stage(ci, b):
        o = ci * CH

        def idxb(g):
            s = g * L
            x = jnp.clip(xy_v[0, pl.ds(o + s, L)], 0.0, 1.0) * float(HXY - 1)
            y = jnp.clip(xy_v[1, pl.ds(o + s, L)], 0.0, 1.0) * float(HXY - 1)
            u = jnp.clip(xy_v[2, pl.ds(o + s, L)], 0.0, 1.0) * float(HUV - 1)
            v = jnp.clip(xy_v[3, pl.ds(o + s, L)], 0.0, 1.0) * float(HUV - 1)
            xi = jnp.minimum(x.astype(jnp.int32), HXY - 2)
            yi = jnp.minimum(y.astype(jnp.int32), HXY - 2)
            ui = jnp.minimum(u.astype(jnp.int32), HUV - 2)
            vi = jnp.minimum(v.astype(jnp.int32), HUV - 2)
            bxy = xi * HXY + yi
            buv = ui * HUV + vi
            ixy_v[b, 0, pl.ds(s, L)] = bxy
            ixy_v[b, 1, pl.ds(s, L)] = bxy + 1
            ixy_v[b, 2, pl.ds(s, L)] = bxy + HXY
            ixy_v[b, 3, pl.ds(s, L)] = bxy + (HXY + 1)
            iuv_v[b, 0, pl.ds(s, L)] = buv
            iuv_v[b, 1, pl.ds(s, L)] = buv + 1
            iuv_v[b, 2, pl.ds(s, L)] = buv + HUV
            iuv_v[b, 3, pl.ds(s, L)] = buv + (HUV + 1)
            wt_v[b, 0, pl.ds(s, L)] = x - xi.astype(jnp.float32)
            wt_v[b, 1, pl.ds(s, L)] = y - yi.astype(jnp.float32)
            wt_v[b, 2, pl.ds(s, L)] = u - ui.astype(jnp.float32)
            wt_v[b, 3, pl.ds(s, L)] = v - vi.astype(jnp.float32)
        with jax.named_scope("sc_idx"):
            plsc.parallel_loop(0, G, unroll=2)(idxb)

        with jax.named_scope("sc_fire"):
            for k in range(4):
                pltpu.async_copy(txy_hbm.at[ixy_v.at[b, k]], rxy_v.at[b, k],
                                 sems_xy[b])
                pltpu.async_copy(tuv_hbm.at[iuv_v.at[b, k]], ruv_v.at[b, k],
                                 sems_uv[b])

    def consume(ci, b):
        cb = base + ci * CH
        with jax.named_scope("sc_wait"):
            for k in range(4):
                pltpu.make_async_copy(txy_hbm.at[ixy_v.at[b, k]],
                                      rxy_v.at[b, k], sems_xy[b]).wait()
                pltpu.make_async_copy(tuv_hbm.at[iuv_v.at[b, k]],
                                      ruv_v.at[b, k], sems_uv[b]).wait()

        def ib(g):
            s = g * L
            wx = wt_v[b, 0, pl.ds(s, L)]
            wy = wt_v[b, 1, pl.ds(s, L)]
            wu = wt_v[b, 2, pl.ds(s, L)]
            wv = wt_v[b, 3, pl.ds(s, L)]
            pxy = ((1.0 - wx) * (1.0 - wy), (1.0 - wx) * wy,
                   wx * (1.0 - wy), wx * wy)
            puv = ((1.0 - wu) * (1.0 - wv), (1.0 - wu) * wv,
                   wu * (1.0 - wv), wu * wv)
            for j in range(L):
                r = s + j
                lane = jnp.full((L,), j, jnp.int32)
                fxy = sum(
                    rxy_v[b, k, r, pl.ds(0, L)]
                    * jnp.take_along_axis(pxy[k], lane, axis=0,
                                          mode="promise_in_bounds")
                    for k in range(4))
                fuv = sum(
                    ruv_v[b, k, r, pl.ds(0, L)]
                    * jnp.take_along_axis(puv[k], lane, axis=0,
                                          mode="promise_in_bounds")
                    for k in range(4))
                k0_v[r // 8, pl.ds((r % 8) * CP, L)] = fxy * fuv
        with jax.named_scope("sc_interp"):
            plsc.parallel_loop(0, G, unroll=1)(ib)

        with jax.named_scope("sc_k0st"):
            pltpu.sync_copy(k0_v, out_hbm.at[pl.ds(cb // 8, CH // 8)])

    stage(0, 0)

    def chunk2(cj, carry):
        for b in range(2):
            ci = cj * 2 + b

            @pl.when(ci + 1 < NCHUNK)
            def _():
                stage(ci + 1, 1 - b)
            consume(ci, b)
        return carry
    lax.fori_loop(0, NCHUNK // 2, chunk2, 0)


@functools.cache
def _sc_interp():
    return functools.partial(
        pl.kernel,
        out_type=jax.ShapeDtypeStruct((N // 8, 8 * CP), jnp.float32),
        mesh=plsc.VectorSubcoreMesh(core_axis_name="c", subcore_axis_name="s",
                                    num_cores=NC, num_subcores=NS),
        scratch_types=[
            pltpu.VMEM((4, BPW), jnp.float32),
            pltpu.VMEM((2, 4, CH), jnp.int32),
            pltpu.VMEM((2, 4, CH), jnp.int32),
            pltpu.VMEM((2, 4, CH), jnp.float32),
            pltpu.VMEM((2, 4, CH, CP), jnp.float32),
            pltpu.VMEM((2, 4, CH, CP), jnp.float32),
            pltpu.VMEM((CH // 8, 8 * CP), jnp.float32),
            pltpu.SemaphoreType.DMA,
            pltpu.SemaphoreType.DMA,
            pltpu.SemaphoreType.DMA,
            pltpu.SemaphoreType.DMA,
        ],
        compiler_params=pltpu.CompilerParams(needs_layout_passes=False,
                                             use_tc_tiling_on_sc=False,
                                             disable_bounds_checks=True),
    )(_sc_body)


def _mlp_body(x_ref, w1_ref, b1_ref, w2_ref, b2_ref, w3_ref, b3_ref, o_ref):
    xp = x_ref[...].astype(jnp.bfloat16)
    outs = []
    for j in range(8):
        x = xp[:, j * CP:(j + 1) * CP]
        h = jnp.maximum(
            jnp.dot(x, w1_ref[...], preferred_element_type=jnp.float32)
            + b1_ref[...], 0.0)
        h = jnp.maximum(
            jnp.dot(h.astype(jnp.bfloat16), w2_ref[...],
                    preferred_element_type=jnp.float32) + b2_ref[...], 0.0)
        o = (jnp.dot(h.astype(jnp.bfloat16), w3_ref[...],
                     preferred_element_type=jnp.float32) + b3_ref[...])
        outs.append(o)
    o_ref[...] = jax.nn.sigmoid(jnp.concatenate(outs, axis=1))


_mlp = pl.pallas_call(
    _mlp_body,
    grid=(N // BN,),
    in_specs=[
        pl.BlockSpec((BN // 8, 8 * CP), lambda i: (i, 0)),
        pl.BlockSpec((CP, 128), lambda i: (0, 0)),
        pl.BlockSpec((1, 128), lambda i: (0, 0)),
        pl.BlockSpec((128, 128), lambda i: (0, 0)),
        pl.BlockSpec((1, 128), lambda i: (0, 0)),
        pl.BlockSpec((128, 8), lambda i: (0, 0)),
        pl.BlockSpec((1, 8), lambda i: (0, 0)),
    ],
    out_specs=pl.BlockSpec((BN // 8, 64), lambda i: (i, 0)),
    out_shape=jax.ShapeDtypeStruct((N // 8, 64), jnp.float32),
)


def kernel(xyuv, plane_xy, plane_uv, W1, b1, W2, b2, W3, b3):
    xyuv_t = xyuv.T
    txy = jnp.pad(plane_xy.reshape(HXY * HXY, C), ((0, 0), (0, CP - C)))
    tuv = jnp.pad(plane_uv.reshape(HUV * HUV, C), ((0, 0), (0, CP - C)))
    k0 = _sc_interp()(xyuv_t, txy, tuv)
    w1p = jnp.pad(W1, ((0, CP - C), (0, 0))).astype(jnp.bfloat16)
    w3p = jnp.pad(W3, ((0, 0), (0, 8 - 3))).astype(jnp.bfloat16)
    b3p = jnp.pad(b3, (0, 8 - 3))
    out = _mlp(k0, w1p, b1.reshape(1, -1), W2.astype(jnp.bfloat16),
               b2.reshape(1, -1), w3p, b3p.reshape(1, -1))
    return out.reshape(N // 8, 8, 8)[:, :, :3].reshape(N, 3)


# speedup vs baseline: 1.4617x; 1.0016x over previous
import functools

import jax
import jax.numpy as jnp
from jax import lax
from jax.experimental import pallas as pl
from jax.experimental.pallas import tpu as pltpu
from jax.experimental.pallas import tpu_sc as plsc

N = 65536
C = 12
CP = 16
NC, NS, L = 2, 16, 16
NW = NC * NS
BPW = N // NW
CH = 128
NCHUNK = BPW // CH
G = CH // L

HXY = 128
HUV = 256
BN = 4096


def _sc_body(xyuv_hbm, txy_hbm, tuv_hbm, out_hbm,
             xy_v, ixy_v, iuv_v, wt_v, rxy_v, ruv_v, k0_v,
             sxy0, sxy1, suv0, suv1):
    wid = lax.axis_index("s") * NC + lax.axis_index("c")
    base = wid * BPW
    sems_xy = (sxy0, sxy1)
    sems_uv = (suv0, suv1)

    pltpu.sync_copy(xyuv_hbm.at[:, pl.ds(base, BPW)], xy_v)

    def stage(ci, b):
        o = ci * CH

        def idxb(g):
            s = g * L
            x = jnp.clip(xy_v[0, pl.ds(o + s, L)], 0.0, 1.0) * float(HXY - 1)
            y = jnp.clip(xy_v[1, pl.ds(o + s, L)], 0.0, 1.0) * float(HXY - 1)
            u = jnp.clip(xy_v[2, pl.ds(o + s, L)], 0.0, 1.0) * float(HUV - 1)
            v = jnp.clip(xy_v[3, pl.ds(o + s, L)], 0.0, 1.0) * float(HUV - 1)
            xi = jnp.minimum(x.astype(jnp.int32), HXY - 2)
            yi = jnp.minimum(y.astype(jnp.int32), HXY - 2)
            ui = jnp.minimum(u.astype(jnp.int32), HUV - 2)
            vi = jnp.minimum(v.astype(jnp.int32), HUV - 2)
            bxy = xi * HXY + yi
            buv = ui * HUV + vi
            ixy_v[b, 0, pl.ds(s, L)] = bxy
            ixy_v[b, 1, pl.ds(s, L)] = bxy + 1
            ixy_v[b, 2, pl.ds(s, L)] = bxy + HXY
            ixy_v[b, 3, pl.ds(s, L)] = bxy + (HXY + 1)
            iuv_v[b, 0, pl.ds(s, L)] = buv
            iuv_v[b, 1, pl.ds(s, L)] = buv + 1
            iuv_v[b, 2, pl.ds(s, L)] = buv + HUV
            iuv_v[b, 3, pl.ds(s, L)] = buv + (HUV + 1)
            wt_v[b, 0, pl.ds(s, L)] = x - xi.astype(jnp.float32)
            wt_v[b, 1, pl.ds(s, L)] = y - yi.astype(jnp.float32)
            wt_v[b, 2, pl.ds(s, L)] = u - ui.astype(jnp.float32)
            wt_v[b, 3, pl.ds(s, L)] = v - vi.astype(jnp.float32)
        with jax.named_scope("sc_idx"):
            plsc.parallel_loop(0, G, unroll=2)(idxb)

        with jax.named_scope("sc_fire"):
            for k in range(4):
                pltpu.async_copy(txy_hbm.at[ixy_v.at[b, k]], rxy_v.at[b, k],
                                 sems_xy[b])
                pltpu.async_copy(tuv_hbm.at[iuv_v.at[b, k]], ruv_v.at[b, k],
                                 sems_uv[b])

    def consume(ci, b):
        cb = base + ci * CH
        with jax.named_scope("sc_wait"):
            for k in range(4):
                pltpu.make_async_copy(txy_hbm.at[ixy_v.at[b, k]],
                                      rxy_v.at[b, k], sems_xy[b]).wait()
                pltpu.make_async_copy(tuv_hbm.at[iuv_v.at[b, k]],
                                      ruv_v.at[b, k], sems_uv[b]).wait()

        def ib(g):
            s = g * L
            wx = wt_v[b, 0, pl.ds(s, L)]
            wy = wt_v[b, 1, pl.ds(s, L)]
            wu = wt_v[b, 2, pl.ds(s, L)]
            wv = wt_v[b, 3, pl.ds(s, L)]
            pxy = ((1.0 - wx) * (1.0 - wy), (1.0 - wx) * wy,
                   wx * (1.0 - wy), wx * wy)
            puv = ((1.0 - wu) * (1.0 - wv), (1.0 - wu) * wv,
                   wu * (1.0 - wv), wu * wv)
            for j in range(L):
                r = s + j
                lane = jnp.full((L,), j, jnp.int32)
                fxy = sum(
                    rxy_v[b, k, r, pl.ds(0, L)]
                    * jnp.take_along_axis(pxy[k], lane, axis=0,
                                          mode="promise_in_bounds")
                    for k in range(4))
                fuv = sum(
                    ruv_v[b, k, r, pl.ds(0, L)]
                    * jnp.take_along_axis(puv[k], lane, axis=0,
                                          mode="promise_in_bounds")
                    for k in range(4))
                k0_v[r // 8, pl.ds((r % 8) * CP, L)] = fxy * fuv
        with jax.named_scope("sc_interp"):
            plsc.parallel_loop(0, G, unroll=1)(ib)

        with jax.named_scope("sc_k0st"):
            pltpu.sync_copy(k0_v, out_hbm.at[pl.ds(cb // 8, CH // 8)])

    stage(0, 0)

    def chunk2(cj, carry):
        for b in range(2):
            ci = cj * 2 + b

            @pl.when(ci + 1 < NCHUNK)
            def _():
                stage(ci + 1, 1 - b)
            consume(ci, b)
        return carry
    lax.fori_loop(0, NCHUNK // 2, chunk2, 0)


@functools.cache
def _sc_interp():
    return functools.partial(
        pl.kernel,
        out_type=jax.ShapeDtypeStruct((N // 8, 8 * CP), jnp.float32),
        mesh=plsc.VectorSubcoreMesh(core_axis_name="c", subcore_axis_name="s",
                                    num_cores=NC, num_subcores=NS),
        scratch_types=[
            pltpu.VMEM((4, BPW), jnp.float32),
            pltpu.VMEM((2, 4, CH), jnp.int32),
            pltpu.VMEM((2, 4, CH), jnp.int32),
            pltpu.VMEM((2, 4, CH), jnp.float32),
            pltpu.VMEM((2, 4, CH, CP), jnp.float32),
            pltpu.VMEM((2, 4, CH, CP), jnp.float32),
            pltpu.VMEM((CH // 8, 8 * CP), jnp.float32),
            pltpu.SemaphoreType.DMA,
            pltpu.SemaphoreType.DMA,
            pltpu.SemaphoreType.DMA,
            pltpu.SemaphoreType.DMA,
        ],
        compiler_params=pltpu.CompilerParams(needs_layout_passes=False,
                                             use_tc_tiling_on_sc=False,
                                             disable_bounds_checks=True),
    )(_sc_body)


def _mlp_body(x_ref, w1_ref, b1_ref, w2_ref, b2_ref, w3_ref, b3_ref, o_ref):
    xp = x_ref[...].astype(jnp.bfloat16)
    outs = []
    for j in range(8):
        x = xp[:, j * CP:(j + 1) * CP]
        h = jnp.maximum(
            jnp.dot(x, w1_ref[...], preferred_element_type=jnp.float32)
            + b1_ref[...], 0.0)
        h = jnp.maximum(
            jnp.dot(h.astype(jnp.bfloat16), w2_ref[...],
                    preferred_element_type=jnp.float32) + b2_ref[...], 0.0)
        o = (jnp.dot(h.astype(jnp.bfloat16), w3_ref[...],
                     preferred_element_type=jnp.float32) + b3_ref[...])
        outs.append(o)
    o_ref[...] = jax.nn.sigmoid(jnp.concatenate(outs, axis=1))


_mlp = pl.pallas_call(
    _mlp_body,
    grid=(N // BN,),
    in_specs=[
        pl.BlockSpec((BN // 8, 8 * CP), lambda i: (i, 0)),
        pl.BlockSpec((CP, 128), lambda i: (0, 0)),
        pl.BlockSpec((1, 128), lambda i: (0, 0)),
        pl.BlockSpec((128, 128), lambda i: (0, 0)),
        pl.BlockSpec((1, 128), lambda i: (0, 0)),
        pl.BlockSpec((128, 8), lambda i: (0, 0)),
        pl.BlockSpec((1, 8), lambda i: (0, 0)),
    ],
    out_specs=pl.BlockSpec((BN // 8, 64), lambda i: (i, 0)),
    out_shape=jax.ShapeDtypeStruct((N // 8, 64), jnp.float32),
)


def kernel(xyuv, plane_xy, plane_uv, W1, b1, W2, b2, W3, b3):
    xyuv_t = xyuv.T
    txy = jnp.concatenate(
        [plane_xy.reshape(HXY * HXY, C),
         jnp.zeros((HXY * HXY, CP - C), jnp.float32)], axis=1)
    tuv = jnp.concatenate(
        [plane_uv.reshape(HUV * HUV, C),
         jnp.zeros((HUV * HUV, CP - C), jnp.float32)], axis=1)
    k0 = _sc_interp()(xyuv_t, txy, tuv)
    w1p = jnp.pad(W1, ((0, CP - C), (0, 0))).astype(jnp.bfloat16)
    w3p = jnp.pad(W3, ((0, 0), (0, 8 - 3))).astype(jnp.bfloat16)
    b3p = jnp.pad(b3, (0, 8 - 3))
    out = _mlp(k0, w1p, b1.reshape(1, -1), W2.astype(jnp.bfloat16),
               b2.reshape(1, -1), w3p, b3p.reshape(1, -1))
    return out.reshape(N // 8, 8, 8)[:, :, :3].reshape(N, 3)
